# Initial kernel scaffold; baseline (speedup 1.0000x reference)
import jax
import jax.numpy as jnp
from jax.experimental import pallas as pl


def kernel(x, edge_index):
    raise NotImplementedError("write your pallas kernel here")


# speedup vs baseline: 3.5653x; 3.5653x over previous
import jax
import jax.numpy as jnp
from jax import lax
from jax.experimental import pallas as pl
from jax.experimental.pallas import tpu as pltpu
from jax.experimental.pallas import tpu_sc as plsc

N_NODES = 10000
N_EDGES = 160000
D_FEAT = 256
DH = 128

NC = 2
NS = 16
CHUNK = 128
EPT = 10112
NCHUNKS = EPT // CHUNK
E_PAD = NS * EPT

ACC_ROWS = 10112
ZROWS = ACC_ROWS // NS
WROWS = 624
WROWS_LAST = N_NODES - 15 * WROWS


def _sc_body(xs_hbm, src_hbm, dst_hbm, zeros_hbm, out_hbm,
             src_v, dst_v, rows_v, acc, sem):
    c = lax.axis_index("c")
    s = lax.axis_index("s")

    pltpu.sync_copy(zeros_hbm, acc.at[pl.ds(s * ZROWS, ZROWS)])
    plsc.subcore_barrier()

    row_off = c * N_NODES

    def chunk_body(k, carry):
        base = s * EPT + k * CHUNK
        pltpu.sync_copy(src_hbm.at[pl.ds(base, CHUNK)], src_v)
        pltpu.sync_copy(dst_hbm.at[pl.ds(base, CHUNK)], dst_v)
        for j in range(CHUNK // 16):
            sl = pl.ds(j * 16, 16)
            src_v[sl] = src_v[sl] + row_off
        pltpu.async_copy(xs_hbm.at[src_v], rows_v, sem).wait()
        pltpu.sync_copy(rows_v, acc.at[dst_v], add=True)
        return carry

    lax.fori_loop(0, NCHUNKS, chunk_body, 0)
    plsc.subcore_barrier()

    @pl.when(s < NS - 1)
    def _():
        pltpu.sync_copy(acc.at[pl.ds(s * WROWS, WROWS)],
                        out_hbm.at[pl.ds(row_off + s * WROWS, WROWS)])

    @pl.when(s == NS - 1)
    def _():
        pltpu.sync_copy(acc.at[pl.ds(15 * WROWS, WROWS_LAST)],
                        out_hbm.at[pl.ds(row_off + 15 * WROWS, WROWS_LAST)])


def kernel(x, edge_index):
    xs = x.reshape(N_NODES, NC, DH).transpose(1, 0, 2).reshape(NC * N_NODES, DH)
    src = edge_index[0].astype(jnp.int32)
    dst = edge_index[1].astype(jnp.int32)
    pad = E_PAD - N_EDGES
    src_p = jnp.concatenate([src, jnp.zeros((pad,), jnp.int32)])
    dst_p = jnp.concatenate([dst, jnp.full((pad,), N_NODES, jnp.int32)])
    zeros = jnp.zeros((ZROWS, DH), jnp.float32)

    mesh = plsc.VectorSubcoreMesh(core_axis_name="c", subcore_axis_name="s",
                                  num_cores=NC, num_subcores=NS)
    out = pl.kernel(
        _sc_body,
        out_type=jax.ShapeDtypeStruct((NC * N_NODES, DH), jnp.float32),
        mesh=mesh,
        scratch_types=[
            pltpu.VMEM((CHUNK,), jnp.int32),
            pltpu.VMEM((CHUNK,), jnp.int32),
            pltpu.VMEM((CHUNK, DH), jnp.float32),
            pltpu.VMEM_SHARED((ACC_ROWS, DH), jnp.float32),
            pltpu.SemaphoreType.DMA,
        ],
    )(xs, src_p, dst_p, zeros)

    return out.reshape(NC, N_NODES, DH).transpose(1, 0, 2).reshape(N_NODES, D_FEAT)


# speedup vs baseline: 3.7590x; 1.0543x over previous
import jax
import jax.numpy as jnp
from jax import lax
from jax.experimental import pallas as pl
from jax.experimental.pallas import tpu as pltpu
from jax.experimental.pallas import tpu_sc as plsc

N_NODES = 10000
N_EDGES = 160000
D_FEAT = 256
DH = 128

NC = 2
NS = 16
CHUNK = 128
NCHUNKS = 80
EPT = NCHUNKS * CHUNK
E_PAD = NS * EPT

ACC_ROWS = 10112
ZROWS = ACC_ROWS // NS
WROWS = 624
WROWS_LAST = N_NODES - 15 * WROWS


def _sc_body(xs_hbm, src_hbm, dst_hbm, zeros_hbm, out_hbm,
             src_v0, src_v1, dst_v0, dst_v1, rows0, rows1, acc,
             semg0, semg1, semi0, semi1, sems0, sems1):
    c = lax.axis_index("c")
    s = lax.axis_index("s")

    pltpu.sync_copy(zeros_hbm, acc.at[pl.ds(s * ZROWS, ZROWS)])
    plsc.subcore_barrier()

    row_off = c * N_NODES
    ebase = s * EPT

    def idx_load(k, src_v, dst_v, sem):
        base = ebase + k * CHUNK
        pltpu.async_copy(src_hbm.at[pl.ds(base, CHUNK)], src_v, sem)
        pltpu.async_copy(dst_hbm.at[pl.ds(base, CHUNK)], dst_v, sem)
        pltpu.make_async_copy(src_hbm.at[pl.ds(base, CHUNK)], src_v, sem).wait()
        pltpu.make_async_copy(dst_hbm.at[pl.ds(base, CHUNK)], dst_v, sem).wait()
        for j in range(CHUNK // 16):
            sl = pl.ds(j * 16, 16)
            src_v[sl] = src_v[sl] + row_off

    def startg(src_v, buf, sem):
        pltpu.async_copy(xs_hbm.at[src_v], buf, sem)

    def waitg(src_v, buf, sem):
        pltpu.make_async_copy(xs_hbm.at[src_v], buf, sem).wait()

    def scat(dst_v, buf, sem):
        pltpu.async_copy(buf, acc.at[dst_v], sem, add=True)
        pltpu.make_async_copy(buf, acc.at[dst_v], sem).wait()

    idx_load(0, src_v0, dst_v0, semi0)
    startg(src_v0, rows0, semg0)
    idx_load(1, src_v1, dst_v1, semi1)
    startg(src_v1, rows1, semg1)

    def pipe(i, carry):
        k = 2 * i
        waitg(src_v0, rows0, semg0)
        scat(dst_v0, rows0, sems0)
        idx_load(k + 2, src_v0, dst_v0, semi0)
        startg(src_v0, rows0, semg0)
        waitg(src_v1, rows1, semg1)
        scat(dst_v1, rows1, sems1)
        idx_load(k + 3, src_v1, dst_v1, semi1)
        startg(src_v1, rows1, semg1)
        return carry

    lax.fori_loop(0, NCHUNKS // 2 - 1, pipe, 0)
    waitg(src_v0, rows0, semg0)
    scat(dst_v0, rows0, sems0)
    waitg(src_v1, rows1, semg1)
    scat(dst_v1, rows1, sems1)

    plsc.subcore_barrier()

    @pl.when(s < NS - 1)
    def _():
        pltpu.sync_copy(acc.at[pl.ds(s * WROWS, WROWS)],
                        out_hbm.at[pl.ds(row_off + s * WROWS, WROWS)])

    @pl.when(s == NS - 1)
    def _():
        pltpu.sync_copy(acc.at[pl.ds(15 * WROWS, WROWS_LAST)],
                        out_hbm.at[pl.ds(row_off + 15 * WROWS, WROWS_LAST)])


def kernel(x, edge_index):
    xs = x.reshape(N_NODES, NC, DH).transpose(1, 0, 2).reshape(NC * N_NODES, DH)
    src = edge_index[0].astype(jnp.int32)
    dst = edge_index[1].astype(jnp.int32)
    pad = E_PAD - N_EDGES
    src_p = jnp.concatenate([src, jnp.zeros((pad,), jnp.int32)])
    dst_p = jnp.concatenate([dst, jnp.full((pad,), N_NODES, jnp.int32)])
    zeros = jnp.zeros((ZROWS, DH), jnp.float32)

    mesh = plsc.VectorSubcoreMesh(core_axis_name="c", subcore_axis_name="s",
                                  num_cores=NC, num_subcores=NS)
    out = pl.kernel(
        _sc_body,
        out_type=jax.ShapeDtypeStruct((NC * N_NODES, DH), jnp.float32),
        mesh=mesh,
        scratch_types=[
            pltpu.VMEM((CHUNK,), jnp.int32),
            pltpu.VMEM((CHUNK,), jnp.int32),
            pltpu.VMEM((CHUNK,), jnp.int32),
            pltpu.VMEM((CHUNK,), jnp.int32),
            pltpu.VMEM((CHUNK, DH), jnp.float32),
            pltpu.VMEM((CHUNK, DH), jnp.float32),
            pltpu.VMEM_SHARED((ACC_ROWS, DH), jnp.float32),
            pltpu.SemaphoreType.DMA,
            pltpu.SemaphoreType.DMA,
            pltpu.SemaphoreType.DMA,
            pltpu.SemaphoreType.DMA,
            pltpu.SemaphoreType.DMA,
            pltpu.SemaphoreType.DMA,
        ],
    )(xs, src_p, dst_p, zeros)

    return out.reshape(NC, N_NODES, DH).transpose(1, 0, 2).reshape(N_NODES, D_FEAT)
